# Initial kernel scaffold; baseline (speedup 1.0000x reference)
import jax
import jax.numpy as jnp
from jax.experimental import pallas as pl


def kernel(x, shells_weight, shell_index, shells_count):
    raise NotImplementedError("write your pallas kernel here")


# speedup vs baseline: 1.4432x; 1.4432x over previous
import functools

import jax
import jax.numpy as jnp
from jax import lax
from jax.experimental import pallas as pl
from jax.experimental.pallas import tpu as pltpu
from jax.experimental.pallas import tpu_sc as plsc

L = 16
NC, NS = 2, 16
NW = NC * NS
BATCH = 256
NPX = 513 * 257
PC = 4096
NFULL = (NPX - 1) // PC
REM = (NPX - 1) - NFULL * PC
REM_VECS = REM // L
NSH = 257
NSP = 272
RPW = BATCH // NW
EPS = 1e-5


def _body(x_hbm, w_hbm, idx_hbm, cnt_hbm, out_hbm,
          x_buf, w_buf, idx_buf, acc, cnt_buf, rec, out_buf):
    wid = lax.axis_index("s") * NC + lax.axis_index("c")
    row0 = wid * RPW

    zeros = jnp.zeros((L,), jnp.float32)

    def zbody(i, c):
        acc[pl.ds(i * L, L)] = zeros
        return c

    lax.fori_loop(0, (RPW * NSP) // L, zbody, 0)

    def accumulate(nvec):
        def vbody(i, c):
            o = i * L
            wv = w_buf[pl.ds(o, L)]
            iv = idx_buf[pl.ds(o, L)]
            for r in range(RPW):
                xv = x_buf[r, pl.ds(o, L)]
                yv = xv * xv * wv
                plsc.addupdate_scatter(acc, [iv + (r * NSP)], yv)
            return c

        lax.fori_loop(0, nvec, vbody, 0)

    def cbody(c, carry):
        base = pl.multiple_of(c * PC, PC)
        pltpu.sync_copy(x_hbm.at[pl.ds(row0, RPW), pl.ds(base, PC)], x_buf)
        pltpu.sync_copy(w_hbm.at[pl.ds(base, PC)], w_buf)
        pltpu.sync_copy(idx_hbm.at[pl.ds(base, PC)], idx_buf)
        accumulate(PC // L)
        return carry

    lax.fori_loop(0, NFULL, cbody, 0)

    rbase = NFULL * PC
    pltpu.sync_copy(x_hbm.at[pl.ds(row0, RPW), pl.ds(rbase, REM)],
                    x_buf.at[:, pl.ds(0, REM)])
    pltpu.sync_copy(w_hbm.at[pl.ds(rbase, REM)], w_buf.at[pl.ds(0, REM)])
    pltpu.sync_copy(idx_hbm.at[pl.ds(rbase, REM)], idx_buf.at[pl.ds(0, REM)])
    accumulate(REM_VECS)

    pltpu.sync_copy(cnt_hbm, cnt_buf)
    for v in range(NSP // L):
        o = v * L
        rec[pl.ds(o, L)] = 1.0 / (cnt_buf[pl.ds(o, L)] + EPS)
    for r in range(RPW):
        for v in range(NSP // L):
            o = v * L
            out_buf[r, pl.ds(o, L)] = acc[pl.ds(r * NSP + o, L)] * rec[pl.ds(o, L)]
    pltpu.sync_copy(out_buf, out_hbm.at[pl.ds(row0, RPW)])


@jax.jit
def _sc_spectrum(x2, w, idx, cnt):
    mesh = plsc.VectorSubcoreMesh(core_axis_name="c", subcore_axis_name="s")
    f = pl.kernel(
        _body,
        mesh=mesh,
        compiler_params=pltpu.CompilerParams(needs_layout_passes=False),
        out_type=jax.ShapeDtypeStruct((BATCH, NSP), jnp.float32),
        scratch_types=[
            pltpu.VMEM((RPW, PC), jnp.float32),
            pltpu.VMEM((PC,), jnp.float32),
            pltpu.VMEM((PC,), jnp.int32),
            pltpu.VMEM((RPW * NSP,), jnp.float32),
            pltpu.VMEM((NSP,), jnp.float32),
            pltpu.VMEM((NSP,), jnp.float32),
            pltpu.VMEM((RPW, NSP), jnp.float32),
        ],
    )
    return f(x2, w, idx, cnt)


def kernel(x, shells_weight, shell_index, shells_count):
    b, c, h, w_ = x.shape
    x2 = x.reshape(b, h * w_)
    wf = shells_weight.reshape(-1)
    idxf = shell_index.reshape(-1)
    cnt = jnp.concatenate(
        [shells_count, jnp.ones((NSP - NSH,), jnp.float32)])
    out = _sc_spectrum(x2, wf, idxf, cnt)
    out = out[:, :NSH]
    last = x2[:, NPX - 1]
    contrib = (last * last) * wf[NPX - 1] / (shells_count[idxf[NPX - 1]] + EPS)
    out = out.at[:, idxf[NPX - 1]].add(contrib)
    return out.reshape(b, c, NSH)
